# Initial kernel scaffold; baseline (speedup 1.0000x reference)
import jax
import jax.numpy as jnp
from jax.experimental import pallas as pl


def kernel(inputs, table, W, b):
    raise NotImplementedError("write your pallas kernel here")


# speedup vs baseline: 1.3565x; 1.3565x over previous
import functools

import jax
import jax.numpy as jnp
from jax import lax
from jax.experimental import pallas as pl
from jax.experimental.pallas import tpu as pltpu
from jax.experimental.pallas import tpu_sc as plsc

D = 128
B = 4096
L = 20

NC, NS = 2, 16
NW = NC * NS
RPW = B // NW
CHUNK = 4
NCHUNK = RPW // CHUNK
IDXC = CHUNK * L


def _pool_body(idx_hbm, table_hbm, out_hbm, idx_v, rows_v, out_v, sem):
    wid = lax.axis_index("s") * NC + lax.axis_index("c")
    pltpu.sync_copy(idx_hbm.at[wid], idx_v)

    def chunk_body(j, carry):
        pltpu.async_copy(table_hbm.at[idx_v.at[j]], rows_v, sem).wait()
        inv = 1.0 / L
        for r in range(CHUNK):
            for d in range(D // 16):
                acc = rows_v[r * L, pl.ds(d * 16, 16)]
                for s in range(1, L):
                    acc = acc + rows_v[r * L + s, pl.ds(d * 16, 16)]
                out_v[j * CHUNK + r, pl.ds(d * 16, 16)] = acc * inv
        return carry

    lax.fori_loop(0, NCHUNK, chunk_body, 0)
    pltpu.sync_copy(out_v, out_hbm.at[pl.ds(wid * RPW, RPW)])


_pool = pl.kernel(
    _pool_body,
    out_type=jax.ShapeDtypeStruct((B, D), jnp.float32),
    mesh=plsc.VectorSubcoreMesh(core_axis_name="c", subcore_axis_name="s"),
    scratch_types=[
        pltpu.VMEM((NCHUNK, IDXC), jnp.int32),
        pltpu.VMEM((IDXC, D), jnp.float32),
        pltpu.VMEM((RPW, D), jnp.float32),
        pltpu.SemaphoreType.DMA,
    ],
)


BLK = 512


def _proj_body(x_ref, w_ref, b_ref, o_ref):
    y = jnp.dot(x_ref[...], w_ref[...], preferred_element_type=jnp.float32)
    y = y + b_ref[...]
    s = jnp.sum(y * y, axis=1, keepdims=True)
    o_ref[...] = y * lax.rsqrt(s)


_proj = pl.pallas_call(
    _proj_body,
    grid=(B // BLK,),
    in_specs=[
        pl.BlockSpec((BLK, D), lambda i: (i, 0)),
        pl.BlockSpec((D, D), lambda i: (0, 0)),
        pl.BlockSpec((1, D), lambda i: (0, 0)),
    ],
    out_specs=pl.BlockSpec((BLK, D), lambda i: (i, 0)),
    out_shape=jax.ShapeDtypeStruct((B, D), jnp.float32),
)


def kernel(inputs, table, W, b):
    idx = inputs.astype(jnp.int32).reshape(NW, NCHUNK, IDXC)
    pooled = _pool(idx, table)
    return _proj(pooled, W, b.reshape(1, D))
